# baseline (speedup 1.0000x reference)
import functools

import jax
import jax.numpy as jnp
from jax import lax
from jax.experimental import pallas as pl
from jax.experimental.pallas import tpu as pltpu
from jax.experimental.pallas import tpu_sc as plsc

NC = 2
NS = 16
L = 16
NW = NC * NS
BINS = 10


def _sc_histogram(pred_blocks, target, rows_w, chunk_rows):
    n_chunks = rows_w // chunk_rows
    unroll = 2
    mesh = plsc.VectorSubcoreMesh(
        core_axis_name="c", subcore_axis_name="s",
        num_cores=NC, num_subcores=NS)

    @functools.partial(
        pl.kernel,
        out_type=jax.ShapeDtypeStruct((2, NW, BINS * L), jnp.float32),
        mesh=mesh,
        scratch_types=[
            pltpu.VMEM((2 * chunk_rows,), jnp.float32),
            pltpu.VMEM((chunk_rows,), jnp.int32),
            pltpu.VMEM((2 * chunk_rows,), jnp.float32),
            pltpu.VMEM((chunk_rows,), jnp.int32),
            pltpu.VMEM((BINS * L,), jnp.float32),
            pltpu.VMEM((BINS * L,), jnp.float32),
            pltpu.VMEM((BINS * L,), jnp.float32),
            pltpu.VMEM((BINS * L,), jnp.float32),
            pltpu.SemaphoreType.DMA,
            pltpu.SemaphoreType.DMA,
        ],
        compiler_params=pltpu.CompilerParams(needs_layout_passes=False),
    )
    def hist(pred_hbm, tgt_hbm, out_hbm,
             pa, ta, pb, tb, cnt0, cnt1, ps0, ps1, sema, semb):
        wid = lax.axis_index("c") * NS + lax.axis_index("s")
        lane = lax.iota(jnp.int32, L)
        lane10 = lane * BINS
        zeros = jnp.zeros((L,), jnp.float32)
        ones = jnp.full((L,), 1.0, jnp.float32)
        onef = jnp.full((L,), 1.0, jnp.float32)
        tenf = jnp.full((L,), 10.0, jnp.float32)
        ninef = jnp.full((L,), float(BINS - 1), jnp.float32)

        for b in range(BINS):
            cnt0[pl.ds(b * L, L)] = zeros
            cnt1[pl.ds(b * L, L)] = zeros
            ps0[pl.ds(b * L, L)] = zeros
            ps1[pl.ds(b * L, L)] = zeros

        row_base = wid * rows_w
        bufs = [(pa, ta, sema), (pb, tb, semb)]

        def start(c, buf):
            bp, bt, sem = buf
            base = row_base + c * chunk_rows
            return (
                pltpu.async_copy(
                    pred_hbm.at[pl.ds(2 * base, 2 * chunk_rows)], bp, sem),
                pltpu.async_copy(tgt_hbm.at[pl.ds(base, chunk_rows)], bt, sem),
            )

        pend = start(0, bufs[0])
        for c in range(n_chunks):
            bp, bt, _ = bufs[c % 2]
            nxt = start(c + 1, bufs[(c + 1) % 2]) if c + 1 < n_chunks else None
            for d in pend:
                d.wait()

            @plsc.parallel_loop(0, chunk_rows // 128, 1, unroll=unroll)
            def _blk_body(g, bp=bp, bt=bt):
                pbase = g * (2 * 128)
                tbase = g * 128
                for u in range(8):
                    p0 = bp[pl.ds(pbase + u * L, L)]
                    p1 = bp[pl.ds(pbase + 128 + u * L, L)]
                    t = bt[pl.ds(tbase + u * L, L)]
                    m0 = t == 0
                    g0 = jnp.where(m0, onef - p0, p0)
                    g1 = jnp.where(m0, p1, onef - p1)
                    p = jnp.where(m0, p0, p1)
                    i0 = jnp.minimum(g0 * tenf, ninef).astype(jnp.int32)
                    i1 = jnp.minimum(g1 * tenf, ninef).astype(jnp.int32)
                    a0 = i0 + lane10
                    a1 = i1 + lane10
                    plsc.addupdate_scatter(cnt0, [a0], ones)
                    plsc.addupdate_scatter(ps0, [a0], p)
                    plsc.addupdate_scatter(cnt1, [a1], ones)
                    plsc.addupdate_scatter(ps1, [a1], p)

            pend = nxt

        for b in range(BINS):
            cnt0[pl.ds(b * L, L)] = cnt0[pl.ds(b * L, L)] + cnt1[pl.ds(b * L, L)]
            ps0[pl.ds(b * L, L)] = ps0[pl.ds(b * L, L)] + ps1[pl.ds(b * L, L)]
        pltpu.sync_copy(cnt0, out_hbm.at[0, wid])
        pltpu.sync_copy(ps0, out_hbm.at[1, wid])

    return hist(pred_blocks, target)


def _combine_body(part_ref, out_ref):
    x = part_ref[...]
    cnt_t = jnp.sum(x[0], axis=(0, 1))
    ps_t = jnp.sum(x[1], axis=(0, 1))
    nz = cnt_t > 0.0
    n = jnp.sum(nz.astype(jnp.float32))
    inv = jnp.where(nz, 1.0 / jnp.where(nz, cnt_t, 1.0), 0.0)
    total = jnp.sum(ps_t * inv)
    out_ref[0, 0] = jnp.where(n > 0.0, -total / (2.0 * n), 0.0)


def kernel(pred, target):
    n_rows = pred.shape[0]
    rows_w = n_rows // NW
    chunk_rows = min(rows_w, 8192)
    pred_blocks = pred.reshape(n_rows // 128, 128, 2).transpose(0, 2, 1)
    pred_blocks = pred_blocks.reshape(2 * n_rows)
    partials = _sc_histogram(pred_blocks, target, rows_w, chunk_rows)
    partials = partials.reshape(2, NW, L, BINS)
    out = pl.pallas_call(
        _combine_body,
        out_shape=jax.ShapeDtypeStruct((1, 1), jnp.float32),
        in_specs=[pl.BlockSpec(memory_space=pltpu.VMEM)],
        out_specs=pl.BlockSpec(memory_space=pltpu.SMEM),
    )(partials)
    return out[0, 0]


# speedup vs baseline: 1.1582x; 1.1582x over previous
import functools

import jax
import jax.numpy as jnp
from jax import lax
from jax.experimental import pallas as pl
from jax.experimental.pallas import tpu as pltpu
from jax.experimental.pallas import tpu_sc as plsc

NC = 2
NS = 16
L = 16
NW = NC * NS
BINS = 10


def _sc_histogram(pred_blocks, target, rows_w, chunk_rows):
    n_chunks = rows_w // chunk_rows
    unroll = 2
    mesh = plsc.VectorSubcoreMesh(
        core_axis_name="c", subcore_axis_name="s",
        num_cores=NC, num_subcores=NS)

    @functools.partial(
        pl.kernel,
        out_type=jax.ShapeDtypeStruct((2, NW, BINS * L), jnp.float32),
        mesh=mesh,
        scratch_types=[
            pltpu.VMEM((2 * chunk_rows,), jnp.float32),
            pltpu.VMEM((chunk_rows,), jnp.int32),
            pltpu.VMEM((2 * chunk_rows,), jnp.float32),
            pltpu.VMEM((chunk_rows,), jnp.int32),
            pltpu.VMEM((BINS * L,), jnp.float32),
            pltpu.VMEM((BINS * L,), jnp.float32),
            pltpu.VMEM((BINS * L,), jnp.float32),
            pltpu.VMEM((BINS * L,), jnp.float32),
            pltpu.SemaphoreType.DMA,
            pltpu.SemaphoreType.DMA,
        ],
        compiler_params=pltpu.CompilerParams(needs_layout_passes=False),
    )
    def hist(pred_hbm, tgt_hbm, out_hbm,
             pa, ta, pb, tb, cnt0, cnt1, ps0, ps1, sema, semb):
        wid = lax.axis_index("c") * NS + lax.axis_index("s")
        lane = lax.iota(jnp.int32, L)
        zeros = jnp.zeros((L,), jnp.float32)
        ones = jnp.full((L,), 1.0, jnp.float32)
        onef = jnp.full((L,), 1.0, jnp.float32)
        tenf = jnp.full((L,), 10.0, jnp.float32)
        ninef = jnp.full((L,), float(BINS - 1), jnp.float32)

        for b in range(BINS):
            cnt0[pl.ds(b * L, L)] = zeros
            cnt1[pl.ds(b * L, L)] = zeros
            ps0[pl.ds(b * L, L)] = zeros
            ps1[pl.ds(b * L, L)] = zeros

        row_base = wid * rows_w
        bufs = [(pa, ta, sema), (pb, tb, semb)]

        def start(c, buf):
            bp, bt, sem = buf
            base = row_base + c * chunk_rows
            return (
                pltpu.async_copy(
                    pred_hbm.at[pl.ds(2 * base, 2 * chunk_rows)], bp, sem),
                pltpu.async_copy(tgt_hbm.at[pl.ds(base, chunk_rows)], bt, sem),
            )

        pend = start(0, bufs[0])
        for c in range(n_chunks):
            bp, bt, _ = bufs[c % 2]
            nxt = start(c + 1, bufs[(c + 1) % 2]) if c + 1 < n_chunks else None
            for d in pend:
                d.wait()

            @plsc.parallel_loop(0, chunk_rows // 128, 1, unroll=unroll)
            def _blk_body(g, bp=bp, bt=bt):
                pbase = g * (2 * 128)
                tbase = g * 128
                for u in range(8):
                    p0 = bp[pl.ds(pbase + u * L, L)]
                    p1 = bp[pl.ds(pbase + 128 + u * L, L)]
                    t = bt[pl.ds(tbase + u * L, L)]
                    m0 = t == 0
                    g0 = jnp.where(m0, onef - p0, p0)
                    g1 = jnp.where(m0, p1, onef - p1)
                    p = jnp.where(m0, p0, p1)
                    i0 = jnp.minimum(g0 * tenf, ninef).astype(jnp.int32)
                    i1 = jnp.minimum(g1 * tenf, ninef).astype(jnp.int32)
                    a0 = i0 * L + lane
                    a1 = i1 * L + lane
                    plsc.addupdate_scatter(cnt0, [a0], ones)
                    plsc.addupdate_scatter(ps0, [a0], p)
                    plsc.addupdate_scatter(cnt1, [a1], ones)
                    plsc.addupdate_scatter(ps1, [a1], p)

            pend = nxt

        for b in range(BINS):
            cnt0[pl.ds(b * L, L)] = cnt0[pl.ds(b * L, L)] + cnt1[pl.ds(b * L, L)]
            ps0[pl.ds(b * L, L)] = ps0[pl.ds(b * L, L)] + ps1[pl.ds(b * L, L)]
        pltpu.sync_copy(cnt0, out_hbm.at[0, wid])
        pltpu.sync_copy(ps0, out_hbm.at[1, wid])

    return hist(pred_blocks, target)


def _combine_body(part_ref, out_ref):
    x = part_ref[...]
    cnt_t = jnp.sum(x[0], axis=(0, 2))
    ps_t = jnp.sum(x[1], axis=(0, 2))
    nz = cnt_t > 0.0
    n = jnp.sum(nz.astype(jnp.float32))
    inv = jnp.where(nz, 1.0 / jnp.where(nz, cnt_t, 1.0), 0.0)
    total = jnp.sum(ps_t * inv)
    out_ref[0, 0] = jnp.where(n > 0.0, -total / (2.0 * n), 0.0)


def kernel(pred, target):
    n_rows = pred.shape[0]
    rows_w = n_rows // NW
    chunk_rows = min(rows_w, 8192)
    pred_blocks = pred.reshape(n_rows // 128, 128, 2).transpose(0, 2, 1)
    pred_blocks = pred_blocks.reshape(2 * n_rows)
    partials = _sc_histogram(pred_blocks, target, rows_w, chunk_rows)
    partials = partials.reshape(2, NW, BINS, L)
    out = pl.pallas_call(
        _combine_body,
        out_shape=jax.ShapeDtypeStruct((1, 1), jnp.float32),
        in_specs=[pl.BlockSpec(memory_space=pltpu.VMEM)],
        out_specs=pl.BlockSpec(memory_space=pltpu.SMEM),
    )(partials)
    return out[0, 0]


# speedup vs baseline: 1.2221x; 1.0552x over previous
import functools

import jax
import jax.numpy as jnp
from jax import lax
from jax.experimental import pallas as pl
from jax.experimental.pallas import tpu as pltpu
from jax.experimental.pallas import tpu_sc as plsc

NC = 2
NS = 16
L = 16
NW = NC * NS
BINS = 10


def _sc_histogram(pred_blocks, target, rows_w, chunk_rows):
    n_chunks = rows_w // chunk_rows
    unroll = 2
    mesh = plsc.VectorSubcoreMesh(
        core_axis_name="c", subcore_axis_name="s",
        num_cores=NC, num_subcores=NS)

    @functools.partial(
        pl.kernel,
        out_type=jax.ShapeDtypeStruct((2, NW, BINS * L), jnp.float32),
        mesh=mesh,
        scratch_types=[
            pltpu.VMEM((2 * chunk_rows,), jnp.float32),
            pltpu.VMEM((chunk_rows,), jnp.int32),
            pltpu.VMEM((2 * chunk_rows,), jnp.float32),
            pltpu.VMEM((chunk_rows,), jnp.int32),
            pltpu.VMEM((BINS * L,), jnp.float32),
            pltpu.VMEM((BINS * L,), jnp.float32),
            pltpu.VMEM((BINS * L,), jnp.float32),
            pltpu.VMEM((BINS * L,), jnp.float32),
            pltpu.SemaphoreType.DMA,
            pltpu.SemaphoreType.DMA,
        ],
        compiler_params=pltpu.CompilerParams(needs_layout_passes=False),
    )
    def hist(pred_hbm, tgt_hbm, out_hbm,
             pa, ta, pb, tb, cnt0, cnt1, ps0, ps1, sema, semb):
        wid = lax.axis_index("c") * NS + lax.axis_index("s")
        lane = lax.iota(jnp.int32, L)
        zeros = jnp.zeros((L,), jnp.float32)
        ones = jnp.full((L,), 1.0, jnp.float32)
        onef = jnp.full((L,), 1.0, jnp.float32)
        tenf = jnp.full((L,), 10.0, jnp.float32)
        ninef = jnp.full((L,), float(BINS - 1), jnp.float32)

        for b in range(BINS):
            cnt0[pl.ds(b * L, L)] = zeros
            cnt1[pl.ds(b * L, L)] = zeros
            ps0[pl.ds(b * L, L)] = zeros
            ps1[pl.ds(b * L, L)] = zeros

        row_base = wid * rows_w
        bufs = [(pa, ta, sema), (pb, tb, semb)]

        def start(c, buf):
            bp, bt, sem = buf
            base = row_base + c * chunk_rows
            return (
                pltpu.async_copy(
                    pred_hbm.at[pl.ds(2 * base, 2 * chunk_rows)], bp, sem),
                pltpu.async_copy(tgt_hbm.at[pl.ds(base, chunk_rows)], bt, sem),
            )

        pend = start(0, bufs[0])
        for c in range(n_chunks):
            bp, bt, _ = bufs[c % 2]
            nxt = start(c + 1, bufs[(c + 1) % 2]) if c + 1 < n_chunks else None
            for d in pend:
                d.wait()

            @plsc.parallel_loop(0, chunk_rows // 128, 1, unroll=unroll)
            def _blk_body(g, bp=bp, bt=bt):
                pbase = g * (2 * 128)
                tbase = g * 128
                for u in range(8):
                    p0 = bp[pl.ds(pbase + u * L, L)]
                    p1 = bp[pl.ds(pbase + 128 + u * L, L)]
                    t = bt[pl.ds(tbase + u * L, L)]
                    m0 = t == 0
                    g0 = jnp.where(m0, onef - p0, p0)
                    g1 = jnp.where(m0, p1, onef - p1)
                    p = jnp.where(m0, p0, p1)
                    i0 = jnp.minimum(g0 * tenf, ninef).astype(jnp.int32)
                    i1 = jnp.minimum(g1 * tenf, ninef).astype(jnp.int32)
                    a0 = i0 * L + lane
                    a1 = i1 * L + lane
                    plsc.addupdate_scatter(cnt0, [a0], ones)
                    plsc.addupdate_scatter(ps0, [a0], p)
                    plsc.addupdate_scatter(cnt1, [a1], ones)
                    plsc.addupdate_scatter(ps1, [a1], p)

            pend = nxt

        for b in range(BINS):
            cnt0[pl.ds(b * L, L)] = cnt0[pl.ds(b * L, L)] + cnt1[pl.ds(b * L, L)]
            ps0[pl.ds(b * L, L)] = ps0[pl.ds(b * L, L)] + ps1[pl.ds(b * L, L)]
        pltpu.sync_copy(cnt0, out_hbm.at[0, wid])
        pltpu.sync_copy(ps0, out_hbm.at[1, wid])

    return hist(pred_blocks, target)


def _combine_body(part_ref, out_ref):
    x = part_ref[...]
    cnt_t = jnp.sum(x[0], axis=(0, 2))
    ps_t = jnp.sum(x[1], axis=(0, 2))
    nz = cnt_t > 0.0
    n = jnp.sum(nz.astype(jnp.float32))
    inv = jnp.where(nz, 1.0 / jnp.where(nz, cnt_t, 1.0), 0.0)
    total = jnp.sum(ps_t * inv)
    out_ref[0, 0] = jnp.where(n > 0.0, -total / (2.0 * n), 0.0)


def kernel(pred, target):
    n_rows = pred.shape[0]
    rows_w = n_rows // NW
    chunk_rows = min(rows_w, 16384)
    pred_blocks = pred.reshape(n_rows // 128, 128, 2).transpose(0, 2, 1)
    pred_blocks = pred_blocks.reshape(2 * n_rows)
    partials = _sc_histogram(pred_blocks, target, rows_w, chunk_rows)
    partials = partials.reshape(2, NW, BINS, L)
    out = pl.pallas_call(
        _combine_body,
        out_shape=jax.ShapeDtypeStruct((1, 1), jnp.float32),
        in_specs=[pl.BlockSpec(memory_space=pltpu.VMEM)],
        out_specs=pl.BlockSpec(memory_space=pltpu.SMEM),
    )(partials)
    return out[0, 0]


# speedup vs baseline: 1.3331x; 1.0908x over previous
import functools

import jax
import jax.numpy as jnp
from jax import lax
from jax.experimental import pallas as pl
from jax.experimental.pallas import tpu as pltpu
from jax.experimental.pallas import tpu_sc as plsc

NC = 2
NS = 16
L = 16
NW = NC * NS
BINS = 10


def _sc_histogram(pred_blocks, target, rows_w, chunk_rows):
    n_chunks = rows_w // chunk_rows
    unroll = 2
    mesh = plsc.VectorSubcoreMesh(
        core_axis_name="c", subcore_axis_name="s",
        num_cores=NC, num_subcores=NS)

    @functools.partial(
        pl.kernel,
        out_type=jax.ShapeDtypeStruct((2, NW, BINS * L), jnp.float32),
        mesh=mesh,
        scratch_types=[
            pltpu.VMEM((2 * chunk_rows,), jnp.float32),
            pltpu.VMEM((chunk_rows,), jnp.int32),
            pltpu.VMEM((2 * chunk_rows,), jnp.float32),
            pltpu.VMEM((chunk_rows,), jnp.int32),
            pltpu.VMEM((BINS * L,), jnp.float32),
            pltpu.VMEM((BINS * L,), jnp.float32),
            pltpu.VMEM((BINS * L,), jnp.float32),
            pltpu.VMEM((BINS * L,), jnp.float32),
            pltpu.SemaphoreType.DMA,
            pltpu.SemaphoreType.DMA,
        ],
        compiler_params=pltpu.CompilerParams(needs_layout_passes=False),
    )
    def hist(pred_hbm, tgt_hbm, out_hbm,
             pa, ta, pb, tb, cnt0, cnt1, ps0, ps1, sema, semb):
        wid = lax.axis_index("c") * NS + lax.axis_index("s")
        lane = lax.iota(jnp.int32, L)
        zeros = jnp.zeros((L,), jnp.float32)
        ones = jnp.full((L,), 1.0, jnp.float32)
        onef = jnp.full((L,), 1.0, jnp.float32)
        tenf = jnp.full((L,), 10.0, jnp.float32)
        ninef = jnp.full((L,), float(BINS - 1), jnp.float32)

        for b in range(BINS):
            cnt0[pl.ds(b * L, L)] = zeros
            cnt1[pl.ds(b * L, L)] = zeros
            ps0[pl.ds(b * L, L)] = zeros
            ps1[pl.ds(b * L, L)] = zeros

        row_base = wid * rows_w
        bufs = [(pa, ta, sema), (pb, tb, semb)]

        def start(c, buf):
            bp, bt, sem = buf
            base = row_base + c * chunk_rows
            return (
                pltpu.async_copy(
                    pred_hbm.at[pl.ds(2 * base, 2 * chunk_rows)], bp, sem),
                pltpu.async_copy(tgt_hbm.at[pl.ds(base, chunk_rows)], bt, sem),
            )

        def wait(buf):
            bp, bt, sem = buf
            pltpu.make_async_copy(
                pred_hbm.at[pl.ds(0, 2 * chunk_rows)], bp, sem).wait()
            pltpu.make_async_copy(
                tgt_hbm.at[pl.ds(0, chunk_rows)], bt, sem).wait()

        def process(buf):
            bp, bt, _ = buf

            @plsc.parallel_loop(0, chunk_rows // 128, 1, unroll=unroll)
            def _blk_body(g):
                pbase = g * (2 * 128)
                tbase = g * 128
                for u in range(8):
                    p0 = bp[pl.ds(pbase + u * L, L)]
                    p1 = bp[pl.ds(pbase + 128 + u * L, L)]
                    t = bt[pl.ds(tbase + u * L, L)]
                    m0 = t == 0
                    g0 = jnp.where(m0, onef - p0, p0)
                    g1 = jnp.where(m0, p1, onef - p1)
                    p = jnp.where(m0, p0, p1)
                    i0 = jnp.minimum(g0 * tenf, ninef).astype(jnp.int32)
                    i1 = jnp.minimum(g1 * tenf, ninef).astype(jnp.int32)
                    a0 = i0 * L + lane
                    a1 = i1 * L + lane
                    plsc.addupdate_scatter(cnt0, [a0], ones)
                    plsc.addupdate_scatter(ps0, [a0], p)
                    plsc.addupdate_scatter(cnt1, [a1], ones)
                    plsc.addupdate_scatter(ps1, [a1], p)

        n_pairs = n_chunks // 2
        start(0, bufs[0])
        start(1, bufs[1])

        def pair_body(cp, _):
            wait(bufs[0])
            process(bufs[0])

            @pl.when(cp + 1 < n_pairs)
            def _():
                start(2 * cp + 2, bufs[0])

            wait(bufs[1])
            process(bufs[1])

            @pl.when(cp + 1 < n_pairs)
            def _():
                start(2 * cp + 3, bufs[1])

            return 0

        lax.fori_loop(0, n_pairs, pair_body, 0)

        for b in range(BINS):
            cnt0[pl.ds(b * L, L)] = cnt0[pl.ds(b * L, L)] + cnt1[pl.ds(b * L, L)]
            ps0[pl.ds(b * L, L)] = ps0[pl.ds(b * L, L)] + ps1[pl.ds(b * L, L)]
        pltpu.sync_copy(cnt0, out_hbm.at[0, wid])
        pltpu.sync_copy(ps0, out_hbm.at[1, wid])

    return hist(pred_blocks, target)


def _combine_body(part_ref, out_ref):
    x = part_ref[...]
    cnt = jnp.sum(x[0], axis=0, keepdims=True)
    ps = jnp.sum(x[1], axis=0, keepdims=True)
    total = jnp.float32(0.0)
    n = jnp.float32(0.0)
    for b in range(BINS):
        cb = jnp.sum(cnt[:, b * L:(b + 1) * L])
        sb = jnp.sum(ps[:, b * L:(b + 1) * L])
        nzb = cb > 0.0
        n = n + nzb.astype(jnp.float32)
        total = total + jnp.where(nzb, sb / jnp.where(nzb, cb, 1.0), 0.0)
    out_ref[0, 0] = jnp.where(n > 0.0, -total / (2.0 * n), 0.0)


def kernel(pred, target):
    n_rows = pred.shape[0]
    rows_w = n_rows // NW
    chunk_rows = min(rows_w, 16384)
    pred_blocks = pred.reshape(n_rows // 128, 128, 2).transpose(0, 2, 1)
    pred_blocks = pred_blocks.reshape(2 * n_rows)
    partials = _sc_histogram(pred_blocks, target, rows_w, chunk_rows)
    out = pl.pallas_call(
        _combine_body,
        out_shape=jax.ShapeDtypeStruct((1, 1), jnp.float32),
        in_specs=[pl.BlockSpec(memory_space=pltpu.VMEM)],
        out_specs=pl.BlockSpec(memory_space=pltpu.SMEM),
    )(partials)
    return out[0, 0]


# speedup vs baseline: 1.3536x; 1.0153x over previous
import functools

import jax
import jax.numpy as jnp
from jax import lax
from jax.experimental import pallas as pl
from jax.experimental.pallas import tpu as pltpu
from jax.experimental.pallas import tpu_sc as plsc

NC = 2
NS = 16
L = 16
NW = NC * NS
BINS = 10


def _sc_histogram(pred_blocks, target, rows_w, chunk_rows):
    n_chunks = rows_w // chunk_rows
    unroll = 4
    mesh = plsc.VectorSubcoreMesh(
        core_axis_name="c", subcore_axis_name="s",
        num_cores=NC, num_subcores=NS)

    @functools.partial(
        pl.kernel,
        out_type=jax.ShapeDtypeStruct((2, NW, BINS * L), jnp.float32),
        mesh=mesh,
        scratch_types=[
            pltpu.VMEM((2 * chunk_rows,), jnp.float32),
            pltpu.VMEM((chunk_rows,), jnp.int32),
            pltpu.VMEM((2 * chunk_rows,), jnp.float32),
            pltpu.VMEM((chunk_rows,), jnp.int32),
            pltpu.VMEM((BINS * L,), jnp.float32),
            pltpu.VMEM((BINS * L,), jnp.float32),
            pltpu.VMEM((BINS * L,), jnp.float32),
            pltpu.VMEM((BINS * L,), jnp.float32),
            pltpu.SemaphoreType.DMA,
            pltpu.SemaphoreType.DMA,
        ],
        compiler_params=pltpu.CompilerParams(needs_layout_passes=False),
    )
    def hist(pred_hbm, tgt_hbm, out_hbm,
             pa, ta, pb, tb, cnt0, cnt1, ps0, ps1, sema, semb):
        wid = lax.axis_index("c") * NS + lax.axis_index("s")
        lane = lax.iota(jnp.int32, L)
        zeros = jnp.zeros((L,), jnp.float32)
        ones = jnp.full((L,), 1.0, jnp.float32)
        onef = jnp.full((L,), 1.0, jnp.float32)
        tenf = jnp.full((L,), 10.0, jnp.float32)
        ninef = jnp.full((L,), float(BINS - 1), jnp.float32)

        for b in range(BINS):
            cnt0[pl.ds(b * L, L)] = zeros
            cnt1[pl.ds(b * L, L)] = zeros
            ps0[pl.ds(b * L, L)] = zeros
            ps1[pl.ds(b * L, L)] = zeros

        row_base = wid * rows_w
        bufs = [(pa, ta, sema), (pb, tb, semb)]

        def start(c, buf):
            bp, bt, sem = buf
            base = row_base + c * chunk_rows
            return (
                pltpu.async_copy(
                    pred_hbm.at[pl.ds(2 * base, 2 * chunk_rows)], bp, sem),
                pltpu.async_copy(tgt_hbm.at[pl.ds(base, chunk_rows)], bt, sem),
            )

        def wait(buf):
            bp, bt, sem = buf
            pltpu.make_async_copy(
                pred_hbm.at[pl.ds(0, 2 * chunk_rows)], bp, sem).wait()
            pltpu.make_async_copy(
                tgt_hbm.at[pl.ds(0, chunk_rows)], bt, sem).wait()

        def process(buf):
            bp, bt, _ = buf

            @plsc.parallel_loop(0, chunk_rows // 128, 1, unroll=unroll)
            def _blk_body(g):
                pbase = g * (2 * 128)
                tbase = g * 128
                for u in range(8):
                    p0 = bp[pl.ds(pbase + u * L, L)]
                    p1 = bp[pl.ds(pbase + 128 + u * L, L)]
                    t = bt[pl.ds(tbase + u * L, L)]
                    m0 = t == 0
                    g0 = jnp.where(m0, onef - p0, p0)
                    g1 = jnp.where(m0, p1, onef - p1)
                    p = jnp.where(m0, p0, p1)
                    i0 = jnp.minimum(g0 * tenf, ninef).astype(jnp.int32)
                    i1 = jnp.minimum(g1 * tenf, ninef).astype(jnp.int32)
                    a0 = i0 * L + lane
                    a1 = i1 * L + lane
                    plsc.addupdate_scatter(cnt0, [a0], ones)
                    plsc.addupdate_scatter(ps0, [a0], p)
                    plsc.addupdate_scatter(cnt1, [a1], ones)
                    plsc.addupdate_scatter(ps1, [a1], p)

        n_pairs = n_chunks // 2
        start(0, bufs[0])
        start(1, bufs[1])

        def pair_body(cp, _):
            wait(bufs[0])
            process(bufs[0])

            @pl.when(cp + 1 < n_pairs)
            def _():
                start(2 * cp + 2, bufs[0])

            wait(bufs[1])
            process(bufs[1])

            @pl.when(cp + 1 < n_pairs)
            def _():
                start(2 * cp + 3, bufs[1])

            return 0

        lax.fori_loop(0, n_pairs, pair_body, 0)

        for b in range(BINS):
            cnt0[pl.ds(b * L, L)] = cnt0[pl.ds(b * L, L)] + cnt1[pl.ds(b * L, L)]
            ps0[pl.ds(b * L, L)] = ps0[pl.ds(b * L, L)] + ps1[pl.ds(b * L, L)]
        pltpu.sync_copy(cnt0, out_hbm.at[0, wid])
        pltpu.sync_copy(ps0, out_hbm.at[1, wid])

    return hist(pred_blocks, target)


def _combine_body(part_ref, out_ref):
    x = part_ref[...]
    cnt = jnp.sum(x[0], axis=0, keepdims=True)
    ps = jnp.sum(x[1], axis=0, keepdims=True)
    total = jnp.float32(0.0)
    n = jnp.float32(0.0)
    for b in range(BINS):
        cb = jnp.sum(cnt[:, b * L:(b + 1) * L])
        sb = jnp.sum(ps[:, b * L:(b + 1) * L])
        nzb = cb > 0.0
        n = n + nzb.astype(jnp.float32)
        total = total + jnp.where(nzb, sb / jnp.where(nzb, cb, 1.0), 0.0)
    out_ref[0, 0] = jnp.where(n > 0.0, -total / (2.0 * n), 0.0)


def kernel(pred, target):
    n_rows = pred.shape[0]
    rows_w = n_rows // NW
    chunk_rows = min(rows_w, 16384)
    pred_blocks = pred.reshape(n_rows // 128, 128, 2).transpose(0, 2, 1)
    pred_blocks = pred_blocks.reshape(2 * n_rows)
    partials = _sc_histogram(pred_blocks, target, rows_w, chunk_rows)
    out = pl.pallas_call(
        _combine_body,
        out_shape=jax.ShapeDtypeStruct((1, 1), jnp.float32),
        in_specs=[pl.BlockSpec(memory_space=pltpu.VMEM)],
        out_specs=pl.BlockSpec(memory_space=pltpu.SMEM),
    )(partials)
    return out[0, 0]
